# baseline (speedup 1.0000x reference)
import functools

import jax
import jax.numpy as jnp
from jax.experimental import pallas as pl
from jax.experimental.pallas import tpu as pltpu

B, N, DIN, DOUT = 4, 2048, 128, 128
BN = 512


def _gcn_body(x_ref, w_ref, adj_ref, bias_ref, out_ref, sup_ref):
    j = pl.program_id(1)

    @pl.when(j == 0)
    def _():
        sup_ref[...] = jnp.dot(
            x_ref[0], w_ref[...], preferred_element_type=jnp.float32
        )

    acc = jax.lax.dot_general(
        adj_ref[0],
        sup_ref[...],
        (((0,), (0,)), ((), ())),
        preferred_element_type=jnp.float32,
    )
    out_ref[0] = acc + bias_ref[...]


@jax.jit
def kernel(input, adj, weight, bias):
    bias2d = bias.reshape(1, DOUT)
    grid = (B, N // BN)
    return pl.pallas_call(
        _gcn_body,
        grid=grid,
        in_specs=[
            pl.BlockSpec((1, N, DIN), lambda b, j: (b, 0, 0)),
            pl.BlockSpec((DIN, DOUT), lambda b, j: (0, 0)),
            pl.BlockSpec((1, N, BN), lambda b, j: (b, 0, j)),
            pl.BlockSpec((1, DOUT), lambda b, j: (0, 0)),
        ],
        out_specs=pl.BlockSpec((1, BN, DOUT), lambda b, j: (b, j, 0)),
        out_shape=jax.ShapeDtypeStruct((B, N, DOUT), jnp.float32),
        scratch_shapes=[pltpu.VMEM((N, DOUT), jnp.float32)],
        compiler_params=pltpu.CompilerParams(
            dimension_semantics=("arbitrary", "arbitrary"),
        ),
    )(input, weight, adj, bias2d)


# speedup vs baseline: 1.0586x; 1.0586x over previous
import functools

import jax
import jax.numpy as jnp
from jax.experimental import pallas as pl
from jax.experimental.pallas import tpu as pltpu

B, N, DIN, DOUT = 4, 2048, 128, 128
BN = 512


def _gcn_body(x_ref, w_ref, adj_ref, bias_ref, out_ref, sup_ref):
    j = pl.program_id(1)

    @pl.when(j == 0)
    def _():
        sup_ref[...] = jnp.dot(
            x_ref[0], w_ref[...], preferred_element_type=jnp.float32
        ).astype(jnp.bfloat16)

    acc = jax.lax.dot_general(
        adj_ref[0].astype(jnp.bfloat16),
        sup_ref[...],
        (((0,), (0,)), ((), ())),
        preferred_element_type=jnp.float32,
    )
    out_ref[0] = acc + bias_ref[...]


@jax.jit
def kernel(input, adj, weight, bias):
    bias2d = bias.reshape(1, DOUT)
    grid = (B, N // BN)
    return pl.pallas_call(
        _gcn_body,
        grid=grid,
        in_specs=[
            pl.BlockSpec((1, N, DIN), lambda b, j: (b, 0, 0)),
            pl.BlockSpec((DIN, DOUT), lambda b, j: (0, 0)),
            pl.BlockSpec((1, N, BN), lambda b, j: (b, 0, j)),
            pl.BlockSpec((1, DOUT), lambda b, j: (0, 0)),
        ],
        out_specs=pl.BlockSpec((1, BN, DOUT), lambda b, j: (b, j, 0)),
        out_shape=jax.ShapeDtypeStruct((B, N, DOUT), jnp.float32),
        scratch_shapes=[pltpu.VMEM((N, DOUT), jnp.bfloat16)],
        compiler_params=pltpu.CompilerParams(
            dimension_semantics=("arbitrary", "arbitrary"),
        ),
    )(input, weight, adj, bias2d)
